# baseline (speedup 1.0000x reference)
import functools

import jax
import jax.numpy as jnp
from jax import lax
from jax.experimental import pallas as pl
from jax.experimental.pallas import tpu as pltpu
from jax.experimental.pallas import tpu_sc as plsc

N = 10000
E = 320000
D = 128
D_OUT = 64

NC, NS = 2, 16
NW = NC * NS
EW = E // NW
C = 125
NCHUNK = EW // C
NPAD = 10240
RT = NPAD // NS

R = 5000
G = N // R


def _sc_mesh():
    return plsc.VectorSubcoreMesh(core_axis_name="c", subcore_axis_name="s")


def _deg_call(dst3, ones_c, zeros_deg):
    @functools.partial(
        pl.kernel,
        mesh=_sc_mesh(),
        out_type=[jax.ShapeDtypeStruct((NPAD, 1), jnp.float32),
                  jax.ShapeDtypeStruct((NPAD, 1), jnp.float32)],
        scratch_types=[
            pltpu.VMEM((NCHUNK, C), jnp.int32),
            pltpu.VMEM((C, 1), jnp.float32),
            pltpu.SemaphoreType.DMA,
            pltpu.VMEM_SHARED((NPAD, 1), jnp.float32),
        ],
    )
    def deg_k(dst_hbm, ones_hbm, zeros_hbm, out0_hbm, out1_hbm,
              dst_all, ones_v, sem, acc_sh):
        cid = lax.axis_index("c")
        sid = lax.axis_index("s")
        w = cid * NS + sid
        pltpu.sync_copy(zeros_hbm, acc_sh.at[pl.ds(sid * RT, RT)])
        pltpu.sync_copy(ones_hbm, ones_v)
        pltpu.sync_copy(dst_hbm.at[w], dst_all)
        plsc.subcore_barrier()

        WDEPTH = 4
        for k in range(WDEPTH):
            pltpu.async_copy(ones_v, acc_sh.at[dst_all.at[k]], sem, add=True)

        def body(i, carry):
            pltpu.make_async_copy(ones_v, acc_sh.at[dst_all.at[0]], sem).wait()
            pltpu.async_copy(ones_v, acc_sh.at[dst_all.at[i + WDEPTH]], sem,
                             add=True)
            return carry

        lax.fori_loop(0, NCHUNK - WDEPTH, body, 0)
        for k in range(WDEPTH):
            pltpu.make_async_copy(ones_v, acc_sh.at[dst_all.at[0]], sem).wait()
        plsc.subcore_barrier()

        @pl.when(cid == 0)
        def _():
            pltpu.sync_copy(acc_sh.at[pl.ds(sid * RT, RT)],
                            out0_hbm.at[pl.ds(sid * RT, RT)])

        @pl.when(cid == 1)
        def _():
            pltpu.sync_copy(acc_sh.at[pl.ds(sid * RT, RT)],
                            out1_hbm.at[pl.ds(sid * RT, RT)])

    return deg_k(dst3, ones_c, zeros_deg)


def _prop_call(v, src3, dst3):
    @functools.partial(
        pl.kernel,
        mesh=_sc_mesh(),
        out_type=jax.ShapeDtypeStruct((NC, NPAD, D), jnp.float32),
        scratch_types=[
            pltpu.VMEM((NCHUNK, C), jnp.int32),
            pltpu.VMEM((C,), jnp.int32),
            pltpu.VMEM((C,), jnp.int32),
            pltpu.VMEM((C, D), jnp.float32),
            pltpu.VMEM((C, D), jnp.float32),
            pltpu.SemaphoreType.DMA,
            pltpu.SemaphoreType.DMA,
            pltpu.SemaphoreType.DMA,
            pltpu.SemaphoreType.DMA,
            pltpu.VMEM_SHARED((NPAD, D), jnp.float32),
        ],
    )
    def prop_k(v_hbm, src_hbm, dst_hbm, out_hbm,
               src_all, d0, d1, rows0, rows1,
               semg0, semg1, semd0, semd1, acc_sh):
        cid = lax.axis_index("c")
        sid = lax.axis_index("s")
        w = cid * NS + sid

        zvec = jnp.zeros((16,), jnp.float32)

        def zbody(a, carry):
            for b in range(D // 16):
                rows0[a, pl.ds(b * 16, 16)] = zvec
            return carry

        lax.fori_loop(0, 120, zbody, 0)
        for off in range(0, 600, 120):
            pltpu.sync_copy(rows0.at[pl.ds(0, 120)],
                            acc_sh.at[pl.ds(sid * RT + off, 120)])
        pltpu.sync_copy(rows0.at[pl.ds(0, 40)],
                        acc_sh.at[pl.ds(sid * RT + 600, 40)])
        pltpu.sync_copy(src_hbm.at[w], src_all)
        plsc.subcore_barrier()

        pltpu.async_copy(v_hbm.at[src_all.at[0]], rows0, semg0)
        pltpu.async_copy(v_hbm.at[src_all.at[1]], rows1, semg1)
        pltpu.async_copy(dst_hbm.at[w, 0], d0, semd0)
        pltpu.async_copy(dst_hbm.at[w, 1], d1, semd1)

        def step(i_cur, i_next, rows, d, semg, semd):
            pltpu.make_async_copy(v_hbm.at[src_all.at[i_cur]], rows, semg).wait()
            pltpu.make_async_copy(dst_hbm.at[0, 0], d, semd).wait()
            pltpu.sync_copy(rows, acc_sh.at[d], add=True)
            pltpu.async_copy(v_hbm.at[src_all.at[i_next]], rows, semg)
            pltpu.async_copy(dst_hbm.at[w, i_next], d, semd)

        def body(j, carry):
            i2 = jnp.minimum(2 * j + 2, NCHUNK - 1)
            i3 = jnp.minimum(2 * j + 3, NCHUNK - 1)
            step(2 * j, i2, rows0, d0, semg0, semd0)
            step(2 * j + 1, i3, rows1, d1, semg1, semd1)
            return carry

        lax.fori_loop(0, NCHUNK // 2, body, 0)
        pltpu.make_async_copy(v_hbm.at[src_all.at[0]], rows0, semg0).wait()
        pltpu.make_async_copy(v_hbm.at[src_all.at[0]], rows1, semg1).wait()
        pltpu.make_async_copy(dst_hbm.at[0, 0], d0, semd0).wait()
        pltpu.make_async_copy(dst_hbm.at[0, 0], d1, semd1).wait()
        plsc.subcore_barrier()
        pltpu.sync_copy(acc_sh.at[pl.ds(sid * RT, RT)],
                        out_hbm.at[cid, pl.ds(sid * RT, RT)])

    return prop_k(v, src3, dst3)


def _dis(d0_ref, d1_ref):
    return lax.rsqrt(d0_ref[...] + d1_ref[...] + 1.0)


def _t1_call(x, w1, deg0, deg1):
    def body(x_ref, w_ref, d0_ref, d1_ref, o_ref):
        xs = x_ref[...] * _dis(d0_ref, d1_ref)
        o_ref[...] = jnp.dot(xs, w_ref[...], preferred_element_type=jnp.float32)

    return pl.pallas_call(
        body,
        grid=(G,),
        in_specs=[pl.BlockSpec((R, D), lambda i: (i, 0)),
                  pl.BlockSpec((D, D), lambda i: (0, 0)),
                  pl.BlockSpec((R, 1), lambda i: (i, 0)),
                  pl.BlockSpec((R, 1), lambda i: (i, 0))],
        out_specs=pl.BlockSpec((R, D), lambda i: (i, 0)),
        out_shape=jax.ShapeDtypeStruct((N, D), jnp.float32),
    )(x, w1, deg0, deg1)


def _t2_call(z, v1, deg0, deg1, b1, w2):
    def body(z0_ref, z1_ref, v1_ref, d0_ref, d1_ref, b_ref, w_ref, o_ref):
        dis = _dis(d0_ref, d1_ref)
        p = dis * (z0_ref[0] + z1_ref[0] + v1_ref[...]) + b_ref[...]
        h = jnp.maximum(p, 0.0)
        o_ref[...] = jnp.dot(h * dis, w_ref[...],
                             preferred_element_type=jnp.float32)

    return pl.pallas_call(
        body,
        grid=(G,),
        in_specs=[pl.BlockSpec((1, R, D), lambda i: (0, i, 0)),
                  pl.BlockSpec((1, R, D), lambda i: (1, i, 0)),
                  pl.BlockSpec((R, D), lambda i: (i, 0)),
                  pl.BlockSpec((R, 1), lambda i: (i, 0)),
                  pl.BlockSpec((R, 1), lambda i: (i, 0)),
                  pl.BlockSpec((1, D), lambda i: (0, 0)),
                  pl.BlockSpec((D, D), lambda i: (0, 0))],
        out_specs=pl.BlockSpec((R, D), lambda i: (i, 0)),
        out_shape=jax.ShapeDtypeStruct((N, D), jnp.float32),
    )(z, z, v1, deg0, deg1, b1.reshape(1, D), w2)


def _t3_call(z, v2, deg0, deg1, b2, wfc, bfc):
    def body(z0_ref, z1_ref, v2_ref, d0_ref, d1_ref, b_ref, w_ref, bo_ref,
             o_ref):
        dis = _dis(d0_ref, d1_ref)
        p = dis * (z0_ref[0] + z1_ref[0] + v2_ref[...]) + b_ref[...]
        o_ref[...] = jnp.dot(p, w_ref[...],
                             preferred_element_type=jnp.float32) + bo_ref[...]

    return pl.pallas_call(
        body,
        grid=(G,),
        in_specs=[pl.BlockSpec((1, R, D), lambda i: (0, i, 0)),
                  pl.BlockSpec((1, R, D), lambda i: (1, i, 0)),
                  pl.BlockSpec((R, D), lambda i: (i, 0)),
                  pl.BlockSpec((R, 1), lambda i: (i, 0)),
                  pl.BlockSpec((R, 1), lambda i: (i, 0)),
                  pl.BlockSpec((1, D), lambda i: (0, 0)),
                  pl.BlockSpec((D, D_OUT), lambda i: (0, 0)),
                  pl.BlockSpec((1, D_OUT), lambda i: (0, 0))],
        out_specs=pl.BlockSpec((R, D_OUT), lambda i: (i, 0)),
        out_shape=jax.ShapeDtypeStruct((N, D_OUT), jnp.float32),
    )(z, z, v2, deg0, deg1, b2.reshape(1, D), wfc, bfc.reshape(1, D_OUT))


def kernel(x, edge_index, W1, b1, W2, b2, Wfc, bfc):
    src3 = edge_index[0].reshape(NW, NCHUNK, C)
    dst3 = edge_index[1].reshape(NW, NCHUNK, C)
    ones_c = jnp.ones((C, 1), jnp.float32)
    zeros_deg = jnp.zeros((RT, 1), jnp.float32)

    deg0, deg1 = _deg_call(dst3, ones_c, zeros_deg)

    v1 = _t1_call(x, W1, deg0, deg1)
    z1 = _prop_call(v1, src3, dst3)
    v2 = _t2_call(z1, v1, deg0, deg1, b1, W2)
    z2 = _prop_call(v2, src3, dst3)
    return _t3_call(z2, v2, deg0, deg1, b2, Wfc, bfc)


# speedup vs baseline: 1.0366x; 1.0366x over previous
import functools

import jax
import jax.numpy as jnp
from jax import lax
from jax.experimental import pallas as pl
from jax.experimental.pallas import tpu as pltpu
from jax.experimental.pallas import tpu_sc as plsc

N = 10000
E = 320000
D = 128
D_OUT = 64

NC, NS = 2, 16
NW = NC * NS
EW = E // NW
C = 125
NCHUNK = EW // C
NPAD = 10240
RT = NPAD // NS

R = 5000
G = N // R


def _sc_mesh():
    return plsc.VectorSubcoreMesh(core_axis_name="c", subcore_axis_name="s")


def _deg_call(dst3, ones_c, zeros_deg):
    @functools.partial(
        pl.kernel,
        mesh=_sc_mesh(),
        out_type=jax.ShapeDtypeStruct((NC * NPAD,), jnp.float32),
        scratch_types=[
            pltpu.VMEM((NCHUNK, C), jnp.int32),
            pltpu.VMEM((C,), jnp.float32),
            pltpu.SemaphoreType.DMA,
            pltpu.VMEM_SHARED((NPAD,), jnp.float32),
        ],
    )
    def deg_k(dst_hbm, ones_hbm, zeros_hbm, out_hbm,
              dst_all, ones_v, sem, acc_sh):
        cid = lax.axis_index("c")
        sid = lax.axis_index("s")
        w = cid * NS + sid
        pltpu.sync_copy(zeros_hbm, acc_sh.at[pl.ds(sid * RT, RT)])
        pltpu.sync_copy(ones_hbm, ones_v)
        pltpu.sync_copy(dst_hbm.at[w], dst_all)
        plsc.subcore_barrier()

        WDEPTH = 4
        for k in range(WDEPTH):
            pltpu.async_copy(ones_v, acc_sh.at[dst_all.at[k]], sem, add=True)

        def body(i, carry):
            pltpu.make_async_copy(ones_v, acc_sh.at[dst_all.at[0]], sem).wait()
            pltpu.async_copy(ones_v, acc_sh.at[dst_all.at[i + WDEPTH]], sem,
                             add=True)
            return carry

        lax.fori_loop(0, NCHUNK - WDEPTH, body, 0)
        for k in range(WDEPTH):
            pltpu.make_async_copy(ones_v, acc_sh.at[dst_all.at[0]], sem).wait()
        plsc.subcore_barrier()
        pltpu.sync_copy(acc_sh.at[pl.ds(sid * RT, RT)],
                        out_hbm.at[pl.ds(cid * NPAD + sid * RT, RT)])

    return deg_k(dst3, ones_c, zeros_deg)


def _prop_call(v, src3, dst3):
    @functools.partial(
        pl.kernel,
        mesh=_sc_mesh(),
        out_type=jax.ShapeDtypeStruct((NC, NPAD, D), jnp.float32),
        scratch_types=[
            pltpu.VMEM((NCHUNK, C), jnp.int32),
            pltpu.VMEM((C,), jnp.int32),
            pltpu.VMEM((C,), jnp.int32),
            pltpu.VMEM((C, D), jnp.float32),
            pltpu.VMEM((C, D), jnp.float32),
            pltpu.SemaphoreType.DMA,
            pltpu.SemaphoreType.DMA,
            pltpu.SemaphoreType.DMA,
            pltpu.SemaphoreType.DMA,
            pltpu.VMEM_SHARED((NPAD, D), jnp.float32),
        ],
    )
    def prop_k(v_hbm, src_hbm, dst_hbm, out_hbm,
               src_all, d0, d1, rows0, rows1,
               semg0, semg1, semd0, semd1, acc_sh):
        cid = lax.axis_index("c")
        sid = lax.axis_index("s")
        w = cid * NS + sid

        zvec = jnp.zeros((16,), jnp.float32)

        def zbody(a, carry):
            for b in range(D // 16):
                rows0[a, pl.ds(b * 16, 16)] = zvec
            return carry

        lax.fori_loop(0, 120, zbody, 0)
        for off in range(0, 600, 120):
            pltpu.sync_copy(rows0.at[pl.ds(0, 120)],
                            acc_sh.at[pl.ds(sid * RT + off, 120)])
        pltpu.sync_copy(rows0.at[pl.ds(0, 40)],
                        acc_sh.at[pl.ds(sid * RT + 600, 40)])
        pltpu.sync_copy(src_hbm.at[w], src_all)
        plsc.subcore_barrier()

        pltpu.async_copy(v_hbm.at[src_all.at[0]], rows0, semg0)
        pltpu.async_copy(v_hbm.at[src_all.at[1]], rows1, semg1)
        pltpu.async_copy(dst_hbm.at[w, 0], d0, semd0)
        pltpu.async_copy(dst_hbm.at[w, 1], d1, semd1)

        def step(i_cur, i_next, rows, d, semg, semd):
            pltpu.make_async_copy(v_hbm.at[src_all.at[i_cur]], rows, semg).wait()
            pltpu.make_async_copy(dst_hbm.at[0, 0], d, semd).wait()
            pltpu.sync_copy(rows, acc_sh.at[d], add=True)
            pltpu.async_copy(v_hbm.at[src_all.at[i_next]], rows, semg)
            pltpu.async_copy(dst_hbm.at[w, i_next], d, semd)

        def body(j, carry):
            i2 = jnp.minimum(2 * j + 2, NCHUNK - 1)
            i3 = jnp.minimum(2 * j + 3, NCHUNK - 1)
            step(2 * j, i2, rows0, d0, semg0, semd0)
            step(2 * j + 1, i3, rows1, d1, semg1, semd1)
            return carry

        lax.fori_loop(0, NCHUNK // 2, body, 0)
        pltpu.make_async_copy(v_hbm.at[src_all.at[0]], rows0, semg0).wait()
        pltpu.make_async_copy(v_hbm.at[src_all.at[0]], rows1, semg1).wait()
        pltpu.make_async_copy(dst_hbm.at[0, 0], d0, semd0).wait()
        pltpu.make_async_copy(dst_hbm.at[0, 0], d1, semd1).wait()
        plsc.subcore_barrier()
        pltpu.sync_copy(acc_sh.at[pl.ds(sid * RT, RT)],
                        out_hbm.at[cid, pl.ds(sid * RT, RT)])

    return prop_k(v, src3, dst3)


def _t1_call(x, w1, dis2):
    def body(x_ref, w_ref, dis_ref, o_ref):
        xs = x_ref[...] * dis_ref[...]
        o_ref[...] = jnp.dot(xs, w_ref[...], preferred_element_type=jnp.float32)

    return pl.pallas_call(
        body,
        grid=(G,),
        in_specs=[pl.BlockSpec((R, D), lambda i: (i, 0)),
                  pl.BlockSpec((D, D), lambda i: (0, 0)),
                  pl.BlockSpec((R, 1), lambda i: (i, 0))],
        out_specs=pl.BlockSpec((R, D), lambda i: (i, 0)),
        out_shape=jax.ShapeDtypeStruct((N, D), jnp.float32),
    )(x, w1, dis2)


def _t2_call(z, v1, dis2, b1, w2):
    def body(z0_ref, z1_ref, v1_ref, dis_ref, b_ref, w_ref, o_ref):
        dis = dis_ref[...]
        p = dis * (z0_ref[0] + z1_ref[0] + v1_ref[...]) + b_ref[...]
        h = jnp.maximum(p, 0.0)
        o_ref[...] = jnp.dot(h * dis, w_ref[...],
                             preferred_element_type=jnp.float32)

    return pl.pallas_call(
        body,
        grid=(G,),
        in_specs=[pl.BlockSpec((1, R, D), lambda i: (0, i, 0)),
                  pl.BlockSpec((1, R, D), lambda i: (1, i, 0)),
                  pl.BlockSpec((R, D), lambda i: (i, 0)),
                  pl.BlockSpec((R, 1), lambda i: (i, 0)),
                  pl.BlockSpec((1, D), lambda i: (0, 0)),
                  pl.BlockSpec((D, D), lambda i: (0, 0))],
        out_specs=pl.BlockSpec((R, D), lambda i: (i, 0)),
        out_shape=jax.ShapeDtypeStruct((N, D), jnp.float32),
    )(z, z, v1, dis2, b1.reshape(1, D), w2)


def _t3_call(z, v2, dis2, b2, wfc, bfc):
    def body(z0_ref, z1_ref, v2_ref, dis_ref, b_ref, w_ref, bo_ref, o_ref):
        p = dis_ref[...] * (z0_ref[0] + z1_ref[0] + v2_ref[...]) + b_ref[...]
        o_ref[...] = jnp.dot(p, w_ref[...],
                             preferred_element_type=jnp.float32) + bo_ref[...]

    return pl.pallas_call(
        body,
        grid=(G,),
        in_specs=[pl.BlockSpec((1, R, D), lambda i: (0, i, 0)),
                  pl.BlockSpec((1, R, D), lambda i: (1, i, 0)),
                  pl.BlockSpec((R, D), lambda i: (i, 0)),
                  pl.BlockSpec((R, 1), lambda i: (i, 0)),
                  pl.BlockSpec((1, D), lambda i: (0, 0)),
                  pl.BlockSpec((D, D_OUT), lambda i: (0, 0)),
                  pl.BlockSpec((1, D_OUT), lambda i: (0, 0))],
        out_specs=pl.BlockSpec((R, D_OUT), lambda i: (i, 0)),
        out_shape=jax.ShapeDtypeStruct((N, D_OUT), jnp.float32),
    )(z, z, v2, dis2, b2.reshape(1, D), wfc, bfc.reshape(1, D_OUT))


def kernel(x, edge_index, W1, b1, W2, b2, Wfc, bfc):
    src3 = edge_index[0].reshape(NW, NCHUNK, C)
    dst3 = edge_index[1].reshape(NW, NCHUNK, C)
    ones_c = jnp.ones((C,), jnp.float32)
    zeros_deg = jnp.zeros((RT,), jnp.float32)

    deg2 = _deg_call(dst3, ones_c, zeros_deg)
    deg = deg2[:N] + deg2[NPAD:NPAD + N] + 1.0
    dis2 = lax.rsqrt(deg)[:, None]

    v1 = _t1_call(x, W1, dis2)
    z1 = _prop_call(v1, src3, dst3)
    v2 = _t2_call(z1, v1, dis2, b1, W2)
    z2 = _prop_call(v2, src3, dst3)
    return _t3_call(z2, v2, dis2, b2, Wfc, bfc)


# speedup vs baseline: 1.0428x; 1.0060x over previous
import functools

import jax
import jax.numpy as jnp
from jax import lax
from jax.experimental import pallas as pl
from jax.experimental.pallas import tpu as pltpu
from jax.experimental.pallas import tpu_sc as plsc

N = 10000
E = 320000
D = 128
D_OUT = 64

NC, NS = 2, 16
NW = NC * NS
EW = E // NW
C = 125
NCHUNK = EW // C
NPAD = 10240
RT = NPAD // NS

R = 5000
G = N // R


def _sc_mesh():
    return plsc.VectorSubcoreMesh(core_axis_name="c", subcore_axis_name="s")


def _deg_call(dst3, ones_c, zeros_deg):
    @functools.partial(
        pl.kernel,
        mesh=_sc_mesh(),
        out_type=jax.ShapeDtypeStruct((NC * NPAD,), jnp.float32),
        scratch_types=[
            pltpu.VMEM((NCHUNK, C), jnp.int32),
            pltpu.VMEM((C,), jnp.float32),
            pltpu.SemaphoreType.DMA,
            pltpu.VMEM_SHARED((NPAD,), jnp.float32),
        ],
    )
    def deg_k(dst_hbm, ones_hbm, zeros_hbm, out_hbm,
              dst_all, ones_v, sem, acc_sh):
        cid = lax.axis_index("c")
        sid = lax.axis_index("s")
        w = cid * NS + sid
        pltpu.sync_copy(zeros_hbm, acc_sh.at[pl.ds(sid * RT, RT)])
        pltpu.sync_copy(ones_hbm, ones_v)
        pltpu.sync_copy(dst_hbm.at[w], dst_all)
        plsc.subcore_barrier()

        WDEPTH = 8
        for k in range(WDEPTH):
            pltpu.async_copy(ones_v, acc_sh.at[dst_all.at[k]], sem, add=True)

        def body(i, carry):
            pltpu.make_async_copy(ones_v, acc_sh.at[dst_all.at[0]], sem).wait()
            pltpu.async_copy(ones_v, acc_sh.at[dst_all.at[i + WDEPTH]], sem,
                             add=True)
            return carry

        lax.fori_loop(0, NCHUNK - WDEPTH, body, 0)
        for k in range(WDEPTH):
            pltpu.make_async_copy(ones_v, acc_sh.at[dst_all.at[0]], sem).wait()
        plsc.subcore_barrier()
        pltpu.sync_copy(acc_sh.at[pl.ds(sid * RT, RT)],
                        out_hbm.at[pl.ds(cid * NPAD + sid * RT, RT)])

    return deg_k(dst3, ones_c, zeros_deg)


def _prop_call(v, src3, dst3):
    @functools.partial(
        pl.kernel,
        mesh=_sc_mesh(),
        out_type=jax.ShapeDtypeStruct((NC, NPAD, D), jnp.float32),
        scratch_types=[
            pltpu.VMEM((NCHUNK, C), jnp.int32),
            pltpu.VMEM((C,), jnp.int32),
            pltpu.VMEM((C,), jnp.int32),
            pltpu.VMEM((C, D), jnp.float32),
            pltpu.VMEM((C, D), jnp.float32),
            pltpu.SemaphoreType.DMA,
            pltpu.SemaphoreType.DMA,
            pltpu.SemaphoreType.DMA,
            pltpu.SemaphoreType.DMA,
            pltpu.VMEM_SHARED((NPAD, D), jnp.float32),
        ],
    )
    def prop_k(v_hbm, src_hbm, dst_hbm, out_hbm,
               src_all, d0, d1, rows0, rows1,
               semg0, semg1, semd0, semd1, acc_sh):
        cid = lax.axis_index("c")
        sid = lax.axis_index("s")
        w = cid * NS + sid

        zvec = jnp.zeros((16,), jnp.float32)

        def zbody(a, carry):
            for b in range(D // 16):
                rows0[a, pl.ds(b * 16, 16)] = zvec
            return carry

        lax.fori_loop(0, 120, zbody, 0)
        for off in range(0, 600, 120):
            pltpu.sync_copy(rows0.at[pl.ds(0, 120)],
                            acc_sh.at[pl.ds(sid * RT + off, 120)])
        pltpu.sync_copy(rows0.at[pl.ds(0, 40)],
                        acc_sh.at[pl.ds(sid * RT + 600, 40)])
        pltpu.sync_copy(src_hbm.at[w], src_all)
        plsc.subcore_barrier()

        pltpu.async_copy(v_hbm.at[src_all.at[0]], rows0, semg0)
        pltpu.async_copy(v_hbm.at[src_all.at[1]], rows1, semg1)
        pltpu.async_copy(dst_hbm.at[w, 0], d0, semd0)
        pltpu.async_copy(dst_hbm.at[w, 1], d1, semd1)

        def step(i_cur, i_next, rows, d, semg, semd):
            pltpu.make_async_copy(v_hbm.at[src_all.at[i_cur]], rows, semg).wait()
            pltpu.make_async_copy(dst_hbm.at[0, 0], d, semd).wait()
            pltpu.sync_copy(rows, acc_sh.at[d], add=True)
            pltpu.async_copy(v_hbm.at[src_all.at[i_next]], rows, semg)
            pltpu.async_copy(dst_hbm.at[w, i_next], d, semd)

        def body(j, carry):
            i2 = jnp.minimum(2 * j + 2, NCHUNK - 1)
            i3 = jnp.minimum(2 * j + 3, NCHUNK - 1)
            step(2 * j, i2, rows0, d0, semg0, semd0)
            step(2 * j + 1, i3, rows1, d1, semg1, semd1)
            return carry

        lax.fori_loop(0, NCHUNK // 2, body, 0)
        pltpu.make_async_copy(v_hbm.at[src_all.at[0]], rows0, semg0).wait()
        pltpu.make_async_copy(v_hbm.at[src_all.at[0]], rows1, semg1).wait()
        pltpu.make_async_copy(dst_hbm.at[0, 0], d0, semd0).wait()
        pltpu.make_async_copy(dst_hbm.at[0, 0], d1, semd1).wait()
        plsc.subcore_barrier()
        pltpu.sync_copy(acc_sh.at[pl.ds(sid * RT, RT)],
                        out_hbm.at[cid, pl.ds(sid * RT, RT)])

    return prop_k(v, src3, dst3)


def _t1_call(x, w1, dis2):
    def body(x_ref, w_ref, dis_ref, o_ref):
        xs = x_ref[...] * dis_ref[...]
        o_ref[...] = jnp.dot(xs, w_ref[...], preferred_element_type=jnp.float32)

    return pl.pallas_call(
        body,
        grid=(G,),
        in_specs=[pl.BlockSpec((R, D), lambda i: (i, 0)),
                  pl.BlockSpec((D, D), lambda i: (0, 0)),
                  pl.BlockSpec((R, 1), lambda i: (i, 0))],
        out_specs=pl.BlockSpec((R, D), lambda i: (i, 0)),
        out_shape=jax.ShapeDtypeStruct((N, D), jnp.float32),
    )(x, w1, dis2)


def _t2_call(z, v1, dis2, b1, w2):
    def body(z0_ref, z1_ref, v1_ref, dis_ref, b_ref, w_ref, o_ref):
        dis = dis_ref[...]
        p = dis * (z0_ref[0] + z1_ref[0] + v1_ref[...]) + b_ref[...]
        h = jnp.maximum(p, 0.0)
        o_ref[...] = jnp.dot(h * dis, w_ref[...],
                             preferred_element_type=jnp.float32)

    return pl.pallas_call(
        body,
        grid=(G,),
        in_specs=[pl.BlockSpec((1, R, D), lambda i: (0, i, 0)),
                  pl.BlockSpec((1, R, D), lambda i: (1, i, 0)),
                  pl.BlockSpec((R, D), lambda i: (i, 0)),
                  pl.BlockSpec((R, 1), lambda i: (i, 0)),
                  pl.BlockSpec((1, D), lambda i: (0, 0)),
                  pl.BlockSpec((D, D), lambda i: (0, 0))],
        out_specs=pl.BlockSpec((R, D), lambda i: (i, 0)),
        out_shape=jax.ShapeDtypeStruct((N, D), jnp.float32),
    )(z, z, v1, dis2, b1.reshape(1, D), w2)


def _t3_call(z, v2, dis2, b2, wfc, bfc):
    def body(z0_ref, z1_ref, v2_ref, dis_ref, b_ref, w_ref, bo_ref, o_ref):
        p = dis_ref[...] * (z0_ref[0] + z1_ref[0] + v2_ref[...]) + b_ref[...]
        o_ref[...] = jnp.dot(p, w_ref[...],
                             preferred_element_type=jnp.float32) + bo_ref[...]

    return pl.pallas_call(
        body,
        grid=(G,),
        in_specs=[pl.BlockSpec((1, R, D), lambda i: (0, i, 0)),
                  pl.BlockSpec((1, R, D), lambda i: (1, i, 0)),
                  pl.BlockSpec((R, D), lambda i: (i, 0)),
                  pl.BlockSpec((R, 1), lambda i: (i, 0)),
                  pl.BlockSpec((1, D), lambda i: (0, 0)),
                  pl.BlockSpec((D, D_OUT), lambda i: (0, 0)),
                  pl.BlockSpec((1, D_OUT), lambda i: (0, 0))],
        out_specs=pl.BlockSpec((R, D_OUT), lambda i: (i, 0)),
        out_shape=jax.ShapeDtypeStruct((N, D_OUT), jnp.float32),
    )(z, z, v2, dis2, b2.reshape(1, D), wfc, bfc.reshape(1, D_OUT))


def kernel(x, edge_index, W1, b1, W2, b2, Wfc, bfc):
    src3 = edge_index[0].reshape(NW, NCHUNK, C)
    dst3 = edge_index[1].reshape(NW, NCHUNK, C)
    ones_c = jnp.ones((C,), jnp.float32)
    zeros_deg = jnp.zeros((RT,), jnp.float32)

    deg2 = _deg_call(dst3, ones_c, zeros_deg)
    deg = deg2[:N] + deg2[NPAD:NPAD + N] + 1.0
    dis2 = lax.rsqrt(deg)[:, None]

    v1 = _t1_call(x, W1, dis2)
    z1 = _prop_call(v1, src3, dst3)
    v2 = _t2_call(z1, v1, dis2, b1, W2)
    z2 = _prop_call(v2, src3, dst3)
    return _t3_call(z2, v2, dis2, b2, Wfc, bfc)
